# Initial kernel scaffold; baseline (speedup 1.0000x reference)
import jax
import jax.numpy as jnp
from jax.experimental import pallas as pl


def kernel(x, edge_index, edge_weight, W1, b1, W2, b2):
    raise NotImplementedError("write your pallas kernel here")


# speedup vs baseline: 5.4857x; 5.4857x over previous
import functools

import jax
import jax.numpy as jnp
from jax import lax
from jax.experimental import pallas as pl
from jax.experimental.pallas import tpu as pltpu
from jax.experimental.pallas import tpu_sc as plsc

N_NODES = 10000
DIM = 128
N_EDGES = 320000

NC, NS = 2, 16
NW = NC * NS
CHUNK = 128
N_CHUNKS = N_EDGES // CHUNK
CPW = N_CHUNKS // NW
EXTRA = N_CHUNKS - CPW * NW
ROWS_PER_SUB = 624
VREGS = DIM // 16


def _spmm_body(h_hbm, row_hbm, col_hbm, w_hbm, out_hbm,
               acc, colv, ridxv, wv, rowsv, scaledv, sem):
    c = lax.axis_index("c")
    s = lax.axis_index("s")
    wid = s * NC + c

    def _zero_row(e, _):
        for j in range(VREGS):
            scaledv[e, 16 * j:16 * (j + 1)] = jnp.zeros((16,), jnp.float32)
        return 0
    lax.fori_loop(0, CHUNK, _zero_row, 0)
    base_row = pl.multiple_of(s * ROWS_PER_SUB, 8)
    for k in range(5):
        pltpu.sync_copy(scaledv.at[pl.ds(0, CHUNK), :],
                        acc.at[pl.ds(base_row + CHUNK * k, CHUNK), :])
    plsc.subcore_barrier()

    def _do_chunk(chunk_idx):
        base = chunk_idx * CHUNK
        pltpu.sync_copy(col_hbm.at[pl.ds(base, CHUNK)], colv)
        gather = pltpu.async_copy(h_hbm.at[colv], rowsv, sem)
        pltpu.sync_copy(w_hbm.at[pl.ds(base, CHUNK)], wv)
        pltpu.sync_copy(row_hbm.at[pl.ds(base, CHUNK)], ridxv)
        gather.wait()

        def _scale_group(g, _):
            wg = wv[pl.ds(g * 16, 16)]
            for e in range(16):
                wspl = jnp.broadcast_to(wg[e], (16,))
                idx = g * 16 + e
                for j in range(VREGS):
                    sl = slice(16 * j, 16 * (j + 1))
                    scaledv[idx, sl] = rowsv[idx, sl] * wspl
            return 0
        lax.fori_loop(0, CHUNK // 16, _scale_group, 0)
        pltpu.sync_copy(scaledv, acc.at[ridxv], add=True)

    def _loop_body(i, _):
        _do_chunk(wid * CPW + i)
        return 0
    lax.fori_loop(0, CPW, _loop_body, 0)

    @pl.when(wid < EXTRA)
    def _():
        _do_chunk(NW * CPW + wid)

    plsc.subcore_barrier()

    @pl.when(s < NS - 1)
    def _():
        pltpu.sync_copy(acc.at[pl.ds(base_row, ROWS_PER_SUB), :],
                        out_hbm.at[c, pl.ds(base_row, ROWS_PER_SUB), :])

    @pl.when(s == NS - 1)
    def _():
        last = (NS - 1) * ROWS_PER_SUB
        pltpu.sync_copy(acc.at[pl.ds(last, N_NODES - last), :],
                        out_hbm.at[c, pl.ds(last, N_NODES - last), :])


@functools.partial(jax.jit, static_argnames=())
def _spmm(h, row, col, w):
    mesh = plsc.VectorSubcoreMesh(core_axis_name="c", subcore_axis_name="s")
    run = pl.kernel(
        _spmm_body,
        out_type=jax.ShapeDtypeStruct((NC, N_NODES, DIM), jnp.float32),
        mesh=mesh,
        scratch_types=[
            pltpu.VMEM_SHARED((N_NODES, DIM), jnp.float32),
            pltpu.VMEM((CHUNK,), jnp.int32),
            pltpu.VMEM((CHUNK,), jnp.int32),
            pltpu.VMEM((CHUNK,), jnp.float32),
            pltpu.VMEM((CHUNK, DIM), jnp.float32),
            pltpu.VMEM((CHUNK, DIM), jnp.float32),
            pltpu.SemaphoreType.DMA,
        ],
    )
    return run(h, row, col, w)


ROW_BLK = 1000


def _mm1_body(x_ref, w_ref, b_ref, o_ref):
    o_ref[...] = jnp.dot(x_ref[...], w_ref[...],
                         preferred_element_type=jnp.float32) + b_ref[...]


def _mm1(x, W, b):
    return pl.pallas_call(
        _mm1_body,
        grid=(N_NODES // ROW_BLK,),
        in_specs=[
            pl.BlockSpec((ROW_BLK, DIM), lambda i: (i, 0)),
            pl.BlockSpec((DIM, DIM), lambda i: (0, 0)),
            pl.BlockSpec((1, DIM), lambda i: (0, 0)),
        ],
        out_specs=pl.BlockSpec((ROW_BLK, DIM), lambda i: (i, 0)),
        out_shape=jax.ShapeDtypeStruct((N_NODES, DIM), jnp.float32),
    )(x, W, b.reshape(1, DIM))


def _mm2_body(p_ref, w_ref, b_ref, o_ref):
    r = jnp.maximum(p_ref[0] + p_ref[1], 0.0)
    o_ref[...] = jnp.dot(r, w_ref[...],
                         preferred_element_type=jnp.float32) + b_ref[...]


def _mm2(p, W, b):
    return pl.pallas_call(
        _mm2_body,
        grid=(N_NODES // ROW_BLK,),
        in_specs=[
            pl.BlockSpec((NC, ROW_BLK, DIM), lambda i: (0, i, 0)),
            pl.BlockSpec((DIM, DIM), lambda i: (0, 0)),
            pl.BlockSpec((1, DIM), lambda i: (0, 0)),
        ],
        out_specs=pl.BlockSpec((ROW_BLK, DIM), lambda i: (i, 0)),
        out_shape=jax.ShapeDtypeStruct((N_NODES, DIM), jnp.float32),
    )(p, W, b.reshape(1, DIM))


def _combine_body(p_ref, o_ref):
    o_ref[...] = p_ref[0] + p_ref[1]


def _combine(p):
    return pl.pallas_call(
        _combine_body,
        grid=(N_NODES // ROW_BLK,),
        in_specs=[pl.BlockSpec((NC, ROW_BLK, DIM), lambda i: (0, i, 0))],
        out_specs=pl.BlockSpec((ROW_BLK, DIM), lambda i: (i, 0)),
        out_shape=jax.ShapeDtypeStruct((N_NODES, DIM), jnp.float32),
    )(p)


def kernel(x, edge_index, edge_weight, W1, b1, W2, b2):
    row = edge_index[0].astype(jnp.int32)
    col = edge_index[1].astype(jnp.int32)
    w = edge_weight.astype(jnp.float32)

    h1 = _mm1(x, W1, b1)
    p1 = _spmm(h1, row, col, w)
    h2 = _mm2(p1, W2, b2)
    p2 = _spmm(h2, row, col, w)
    return _combine(p2)


# speedup vs baseline: 10.3585x; 1.8883x over previous
import functools

import jax
import jax.numpy as jnp
from jax import lax
from jax.experimental import pallas as pl
from jax.experimental.pallas import tpu as pltpu
from jax.experimental.pallas import tpu_sc as plsc

N_NODES = 10000
DIM = 128
HDIM = DIM // 2
N_EDGES = 320000

NC, NS = 2, 16
NW = NC * NS
CHUNK = 128
CPW = 80
PAD_EDGES = NW * CPW * CHUNK
ROWS_PER_SUB = 624
HVREGS = HDIM // 16
NBUF = 4


def _spmm_body(ha_hbm, hb_hbm, row_hbm, col_hbm, w_hbm, outa_hbm, outb_hbm,
               acc, colv, ridxv, wv, rows, scaled,
               gsems, ssems):
    c = lax.axis_index("c")
    s = lax.axis_index("s")
    wid = s * NC + c
    base_c = wid * CPW

    pltpu.sync_copy(col_hbm.at[pl.ds(base_c, CPW)], colv)
    pltpu.sync_copy(row_hbm.at[pl.ds(base_c, CPW)], ridxv)
    pltpu.sync_copy(w_hbm.at[pl.ds(base_c, CPW)], wv)

    base_row = pl.multiple_of(s * ROWS_PER_SUB, 8)

    for h_hbm, out_hbm in ((ha_hbm, outa_hbm), (hb_hbm, outb_hbm)):
        for b in range(NBUF):
            pltpu.async_copy(h_hbm.at[colv.at[b]], rows.at[b], gsems.at[b])

        def _zero_row(e, _):
            for j in range(HVREGS):
                scaled[0, e, 16 * j:16 * (j + 1)] = jnp.zeros((16,),
                                                              jnp.float32)
            return 0
        lax.fori_loop(0, CHUNK, _zero_row, 0)
        for k in range(5):
            pltpu.sync_copy(scaled.at[0, pl.ds(0, CHUNK), :],
                            acc.at[pl.ds(base_row + CHUNK * k, CHUNK), :])
        plsc.subcore_barrier()

        def _step(t, b):
            i = NBUF * t + b
            sb = b % 2
            pltpu.make_async_copy(h_hbm.at[colv.at[i]], rows.at[b],
                                  gsems.at[b]).wait()

            @pl.when(i >= 2)
            def _():
                pltpu.make_async_copy(scaled.at[sb], acc.at[ridxv.at[i - 2]],
                                      ssems.at[sb]).wait()

            def _scale_group(g, _):
                wg = wv[i, pl.ds(g * 16, 16)]
                for e in range(16):
                    wspl = jnp.broadcast_to(wg[e], (16,))
                    idx = g * 16 + e
                    for j in range(HVREGS):
                        sl = slice(16 * j, 16 * (j + 1))
                        scaled[sb, idx, sl] = rows[b, idx, sl] * wspl
                return 0
            lax.fori_loop(0, CHUNK // 16, _scale_group, 0)
            pltpu.async_copy(scaled.at[sb], acc.at[ridxv.at[i]],
                             ssems.at[sb], add=True)

            @pl.when(i + NBUF < CPW)
            def _():
                pltpu.async_copy(h_hbm.at[colv.at[i + NBUF]], rows.at[b],
                                 gsems.at[b])

        def _loop_body(t, _):
            for b in range(NBUF):
                _step(t, b)
            return 0
        lax.fori_loop(0, CPW // NBUF, _loop_body, 0)

        for i in (CPW - 2, CPW - 1):
            pltpu.make_async_copy(scaled.at[i % 2], acc.at[ridxv.at[i]],
                                  ssems.at[i % 2]).wait()
        plsc.subcore_barrier()

        @pl.when(s < NS - 1)
        def _():
            pltpu.sync_copy(acc.at[pl.ds(base_row, ROWS_PER_SUB), :],
                            out_hbm.at[c, pl.ds(base_row, ROWS_PER_SUB), :])

        @pl.when(s == NS - 1)
        def _():
            last = (NS - 1) * ROWS_PER_SUB
            pltpu.sync_copy(acc.at[pl.ds(last, N_NODES - last), :],
                            out_hbm.at[c, pl.ds(last, N_NODES - last), :])

        plsc.subcore_barrier()


@jax.jit
def _spmm(ha, hb, row, col, w):
    mesh = plsc.VectorSubcoreMesh(core_axis_name="c", subcore_axis_name="s")
    run = pl.kernel(
        _spmm_body,
        out_type=(jax.ShapeDtypeStruct((NC, N_NODES, HDIM), jnp.float32),
                  jax.ShapeDtypeStruct((NC, N_NODES, HDIM), jnp.float32)),
        mesh=mesh,
        compiler_params=pltpu.CompilerParams(use_tc_tiling_on_sc=False),
        scratch_types=[
            pltpu.VMEM_SHARED((N_NODES, HDIM), jnp.float32),
            pltpu.VMEM((CPW, CHUNK), jnp.int32),
            pltpu.VMEM((CPW, CHUNK), jnp.int32),
            pltpu.VMEM((CPW, CHUNK), jnp.float32),
            pltpu.VMEM((NBUF, CHUNK, HDIM), jnp.float32),
            pltpu.VMEM((2, CHUNK, HDIM), jnp.float32),
            pltpu.SemaphoreType.DMA((NBUF,)),
            pltpu.SemaphoreType.DMA((2,)),
        ],
    )
    return run(ha, hb, row, col, w)


ROW_BLK = 1000


def _mm1_body(x_ref, w_ref, b_ref, oa_ref, ob_ref):
    h = jnp.dot(x_ref[...], w_ref[...],
                preferred_element_type=jnp.float32) + b_ref[...]
    oa_ref[...] = h[:, :HDIM]
    ob_ref[...] = h[:, HDIM:]


def _mm1(x, W, b):
    return pl.pallas_call(
        _mm1_body,
        grid=(N_NODES // ROW_BLK,),
        in_specs=[
            pl.BlockSpec((ROW_BLK, DIM), lambda i: (i, 0)),
            pl.BlockSpec((DIM, DIM), lambda i: (0, 0)),
            pl.BlockSpec((1, DIM), lambda i: (0, 0)),
        ],
        out_specs=[pl.BlockSpec((ROW_BLK, HDIM), lambda i: (i, 0)),
                   pl.BlockSpec((ROW_BLK, HDIM), lambda i: (i, 0))],
        out_shape=[jax.ShapeDtypeStruct((N_NODES, HDIM), jnp.float32),
                   jax.ShapeDtypeStruct((N_NODES, HDIM), jnp.float32)],
    )(x, W, b.reshape(1, DIM))


def _mm2_body(pa_ref, pb_ref, w_ref, b_ref, oa_ref, ob_ref):
    r = jnp.maximum(
        jnp.concatenate([pa_ref[0] + pa_ref[1], pb_ref[0] + pb_ref[1]],
                        axis=1), 0.0)
    h = jnp.dot(r, w_ref[...], preferred_element_type=jnp.float32) + b_ref[...]
    oa_ref[...] = h[:, :HDIM]
    ob_ref[...] = h[:, HDIM:]


def _mm2(pa, pb, W, b):
    return pl.pallas_call(
        _mm2_body,
        grid=(N_NODES // ROW_BLK,),
        in_specs=[
            pl.BlockSpec((NC, ROW_BLK, HDIM), lambda i: (0, i, 0)),
            pl.BlockSpec((NC, ROW_BLK, HDIM), lambda i: (0, i, 0)),
            pl.BlockSpec((DIM, DIM), lambda i: (0, 0)),
            pl.BlockSpec((1, DIM), lambda i: (0, 0)),
        ],
        out_specs=[pl.BlockSpec((ROW_BLK, HDIM), lambda i: (i, 0)),
                   pl.BlockSpec((ROW_BLK, HDIM), lambda i: (i, 0))],
        out_shape=[jax.ShapeDtypeStruct((N_NODES, HDIM), jnp.float32),
                   jax.ShapeDtypeStruct((N_NODES, HDIM), jnp.float32)],
    )(pa, pb, W, b.reshape(1, DIM))


def _combine_body(pa_ref, pb_ref, o_ref):
    o_ref[...] = jnp.concatenate(
        [pa_ref[0] + pa_ref[1], pb_ref[0] + pb_ref[1]], axis=1)


def _combine(pa, pb):
    return pl.pallas_call(
        _combine_body,
        grid=(N_NODES // ROW_BLK,),
        in_specs=[pl.BlockSpec((NC, ROW_BLK, HDIM), lambda i: (0, i, 0)),
                  pl.BlockSpec((NC, ROW_BLK, HDIM), lambda i: (0, i, 0))],
        out_specs=pl.BlockSpec((ROW_BLK, DIM), lambda i: (i, 0)),
        out_shape=jax.ShapeDtypeStruct((N_NODES, DIM), jnp.float32),
    )(pa, pb)


def kernel(x, edge_index, edge_weight, W1, b1, W2, b2):
    pad = PAD_EDGES - N_EDGES
    spread = (jnp.arange(pad, dtype=jnp.int32) * 37) % N_NODES
    row = jnp.concatenate(
        [edge_index[0].astype(jnp.int32), spread]).reshape(-1, CHUNK)
    col = jnp.concatenate(
        [edge_index[1].astype(jnp.int32), spread]).reshape(-1, CHUNK)
    w = jnp.concatenate(
        [edge_weight.astype(jnp.float32),
         jnp.zeros((pad,), jnp.float32)]).reshape(-1, CHUNK)

    h1a, h1b = _mm1(x, W1, b1)
    p1a, p1b = _spmm(h1a, h1b, row, col, w)
    h2a, h2b = _mm2(p1a, p1b, W2, b2)
    p2a, p2b = _spmm(h2a, h2b, row, col, w)
    return _combine(p2a, p2b)


# speedup vs baseline: 11.9829x; 1.1568x over previous
import jax
import jax.numpy as jnp
from jax import lax
from jax.experimental import pallas as pl
from jax.experimental.pallas import tpu as pltpu
from jax.experimental.pallas import tpu_sc as plsc

N_NODES = 10000
DIM = 128
HDIM = DIM // 2
N_EDGES = 320000

NC, NS = 2, 16
CHUNK = 128
CPT = 160
PAD_EDGES = NS * CPT * CHUNK
PHASE = 80
ROWS_PER_SUB = 624
HVREGS = HDIM // 16
NBUF = 4


def _spmm_body(h_hbm, row_hbm, col_hbm, w_hbm, out_hbm,
               acc, colv, ridxv, wv, rows, scaled, gsems, ssems):
    c = lax.axis_index("c")
    s = lax.axis_index("s")

    def _zero_row(e, _):
        for j in range(HVREGS):
            scaled[0, e, 16 * j:16 * (j + 1)] = jnp.zeros((16,), jnp.float32)
        return 0
    lax.fori_loop(0, CHUNK, _zero_row, 0)
    base_row = pl.multiple_of(s * ROWS_PER_SUB, 8)
    for k in range(5):
        pltpu.sync_copy(scaled.at[0, pl.ds(0, CHUNK), :],
                        acc.at[pl.ds(base_row + CHUNK * k, CHUNK), :])
    plsc.subcore_barrier()

    hsrc = h_hbm.at[c]

    for phase in range(CPT // PHASE):
        pbase = s * CPT + phase * PHASE
        pltpu.sync_copy(col_hbm.at[pl.ds(pbase, PHASE)], colv)
        pltpu.sync_copy(row_hbm.at[pl.ds(pbase, PHASE)], ridxv)
        pltpu.sync_copy(w_hbm.at[pl.ds(pbase, PHASE)], wv)

        for b in range(NBUF):
            pltpu.async_copy(hsrc.at[colv.at[b]], rows.at[b], gsems.at[b])

        def _step(t, b):
            i = NBUF * t + b
            sb = b % 2
            pltpu.make_async_copy(hsrc.at[colv.at[i]], rows.at[b],
                                  gsems.at[b]).wait()

            @pl.when(i >= 2)
            def _():
                pltpu.make_async_copy(scaled.at[sb], acc.at[ridxv.at[i - 2]],
                                      ssems.at[sb]).wait()

            def _scale_group(g, _):
                wg = wv[i, pl.ds(g * 16, 16)]
                for e in range(16):
                    wspl = jnp.broadcast_to(wg[e], (16,))
                    idx = g * 16 + e
                    for j in range(HVREGS):
                        sl = slice(16 * j, 16 * (j + 1))
                        scaled[sb, idx, sl] = rows[b, idx, sl] * wspl
                return 0
            lax.fori_loop(0, CHUNK // 16, _scale_group, 0)
            pltpu.async_copy(scaled.at[sb], acc.at[ridxv.at[i]],
                             ssems.at[sb], add=True)

            @pl.when(i + NBUF < PHASE)
            def _():
                pltpu.async_copy(hsrc.at[colv.at[i + NBUF]], rows.at[b],
                                 gsems.at[b])

        def _loop_body(t, _):
            for b in range(NBUF):
                _step(t, b)
            return 0
        lax.fori_loop(0, PHASE // NBUF, _loop_body, 0)

        for i in (PHASE - 2, PHASE - 1):
            pltpu.make_async_copy(scaled.at[i % 2], acc.at[ridxv.at[i]],
                                  ssems.at[i % 2]).wait()

    plsc.subcore_barrier()

    col_base = pl.multiple_of(c * HDIM, 8)

    @pl.when(s < NS - 1)
    def _():
        pltpu.sync_copy(acc.at[pl.ds(base_row, ROWS_PER_SUB), :],
                        out_hbm.at[pl.ds(base_row, ROWS_PER_SUB),
                                   pl.ds(col_base, HDIM)])

    @pl.when(s == NS - 1)
    def _():
        last = (NS - 1) * ROWS_PER_SUB
        pltpu.sync_copy(acc.at[pl.ds(last, N_NODES - last), :],
                        out_hbm.at[pl.ds(last, N_NODES - last),
                                   pl.ds(col_base, HDIM)])


@jax.jit
def _spmm(h2, row, col, w):
    mesh = plsc.VectorSubcoreMesh(core_axis_name="c", subcore_axis_name="s")
    run = pl.kernel(
        _spmm_body,
        out_type=jax.ShapeDtypeStruct((N_NODES, DIM), jnp.float32),
        mesh=mesh,
        compiler_params=pltpu.CompilerParams(use_tc_tiling_on_sc=False),
        scratch_types=[
            pltpu.VMEM_SHARED((N_NODES, HDIM), jnp.float32),
            pltpu.VMEM((PHASE, CHUNK), jnp.int32),
            pltpu.VMEM((PHASE, CHUNK), jnp.int32),
            pltpu.VMEM((PHASE, CHUNK), jnp.float32),
            pltpu.VMEM((NBUF, CHUNK, HDIM), jnp.float32),
            pltpu.VMEM((2, CHUNK, HDIM), jnp.float32),
            pltpu.SemaphoreType.DMA((NBUF,)),
            pltpu.SemaphoreType.DMA((2,)),
        ],
    )
    return run(h2, row, col, w)


ROW_BLK = 2000


def _mm_body(x_ref, w_ref, b_ref, o_ref, *, relu_in):
    xin = x_ref[...]
    if relu_in:
        xin = jnp.maximum(xin, 0.0)
    h = jnp.dot(xin, w_ref[...], preferred_element_type=jnp.float32) \
        + b_ref[...]
    o_ref[0] = h[:, :HDIM]
    o_ref[1] = h[:, HDIM:]


def _mm(x, W, b, relu_in):
    import functools
    return pl.pallas_call(
        functools.partial(_mm_body, relu_in=relu_in),
        grid=(N_NODES // ROW_BLK,),
        in_specs=[
            pl.BlockSpec((ROW_BLK, DIM), lambda i: (i, 0)),
            pl.BlockSpec((DIM, DIM), lambda i: (0, 0)),
            pl.BlockSpec((1, DIM), lambda i: (0, 0)),
        ],
        out_specs=pl.BlockSpec((NC, ROW_BLK, HDIM), lambda i: (0, i, 0)),
        out_shape=jax.ShapeDtypeStruct((NC, N_NODES, HDIM), jnp.float32),
    )(x, W, b.reshape(1, DIM))


def kernel(x, edge_index, edge_weight, W1, b1, W2, b2):
    pad = PAD_EDGES - N_EDGES
    spread = (jnp.arange(pad, dtype=jnp.int32) * 37) % N_NODES
    row = jnp.concatenate(
        [edge_index[0].astype(jnp.int32), spread]).reshape(-1, CHUNK)
    col = jnp.concatenate(
        [edge_index[1].astype(jnp.int32), spread]).reshape(-1, CHUNK)
    w = jnp.concatenate(
        [edge_weight.astype(jnp.float32),
         jnp.zeros((pad,), jnp.float32)]).reshape(-1, CHUNK)

    h1 = _mm(x, W1, b1, relu_in=False)
    p1 = _spmm(h1, row, col, w)
    h2 = _mm(p1, W2, b2, relu_in=True)
    return _spmm(h2, row, col, w)
